# baseline (speedup 1.0000x reference)
import functools

import jax
import jax.numpy as jnp
from jax import lax
from jax.experimental import pallas as pl
from jax.experimental.pallas import tpu as pltpu
from jax.experimental.pallas import tpu_sc as plsc

N = 10000
E = 320000
B = 256
NOUT = 128
EOUT = 16
HS = 256
UOUT = 128

NC = 2
NS = 16
NW = NC * NS
L = 16

EPT = E // NW
CH = 2048
NFULL = EPT // CH
TAIL = EPT - NFULL * CH
GRPS = CH // L
NBUF = 3


def _edge_agg_sc(row16, attr_t, batch_i):
    mesh = plsc.VectorSubcoreMesh(core_axis_name="c", subcore_axis_name="s")

    @functools.partial(
        pl.kernel,
        out_type=[
            jax.ShapeDtypeStruct((NC, B, EOUT), jnp.float32),
            jax.ShapeDtypeStruct((NC, 32, 16), jnp.float32),
        ],
        mesh=mesh,
        compiler_params=pltpu.CompilerParams(needs_layout_passes=False,
                                             use_tc_tiling_on_sc=False),
        scratch_types=[
            pltpu.VMEM((N,), jnp.int32),
            pltpu.VMEM((NBUF, GRPS, L), jnp.int32),
            pltpu.VMEM((NBUF, EOUT, CH), jnp.float32),
            pltpu.VMEM((EOUT, B + 16), jnp.float32),
            pltpu.VMEM((B + 16, EOUT), jnp.float32),
            pltpu.VMEM((32, 16), jnp.float32),
            pltpu.VMEM((B + 16, EOUT), jnp.float32),
            pltpu.VMEM((32,), jnp.int32),
            pltpu.VMEM((2, 128), jnp.int32),
            [pltpu.SemaphoreType.DMA] * NBUF,
            pltpu.VMEM_SHARED((B + 16, EOUT), jnp.float32),
            pltpu.VMEM_SHARED((32, 16), jnp.float32),
        ],
    )
    def k(row_hbm, attr_hbm, batch_hbm, esum_hbm, ecnt_hbm,
          btbl, rows_v, attr_v, acc, accm, cntl, zbuf, i32v, i128v,
          sl, acc_sh, cnt_sh):
        c = lax.axis_index("c")
        s = lax.axis_index("s")
        wid = s * NC + c
        iota16 = lax.iota(jnp.int32, L)

        def zrow(i, carry):
            zbuf[i, :] = jnp.zeros((L,), jnp.float32)
            return carry
        lax.fori_loop(0, B + 16, zrow, 0)

        def arow(i, carry):
            for f in range(EOUT):
                acc[f, pl.ds(i * L, L)] = jnp.zeros((L,), jnp.float32)
            return carry
        lax.fori_loop(0, (B + 16) // L, arow, 0)

        def crow(i, carry):
            cntl[i, :] = jnp.zeros((L,), jnp.float32)
            return carry
        lax.fori_loop(0, 32, crow, 0)

        i32v[pl.ds(0, 16)] = iota16
        i32v[pl.ds(16, 16)] = iota16 + 16

        def irow(i, carry):
            j = lax.shift_right_logical(i, 3)
            o = (i & 7) * L
            i128v[j, pl.ds(o, L)] = iota16 + i * L
            return carry
        lax.fori_loop(0, 16, irow, 0)

        @pl.when(s == 0)
        def _():
            pltpu.sync_copy(zbuf, acc_sh)
            pltpu.sync_copy(zbuf.at[pl.ds(0, 32), :], cnt_sh)

        pltpu.sync_copy(batch_hbm, btbl)
        plsc.subcore_barrier()

        nchunk = NFULL + 1

        def start_loads(ci):
            p = ci % NBUF
            sz = CH if ci < NFULL else TAIL
            off = wid * EPT + ci * CH
            return [
                pltpu.async_copy(row_hbm.at[pl.ds(off // L, sz // L), :],
                                 rows_v.at[p, pl.ds(0, sz // L), :], sl[p]),
            ] + [
                pltpu.async_copy(attr_hbm.at[f, pl.ds(off, sz)],
                                 attr_v.at[p, f, pl.ds(0, sz)], sl[p])
                for f in range(EOUT)
            ]

        loads = {0: start_loads(0)}
        for ci in range(nchunk):
            p = ci % NBUF
            sz = CH if ci < NFULL else TAIL
            if ci + 1 < nchunk:
                loads[ci + 1] = start_loads(ci + 1)
            for d in loads.pop(ci):
                d.wait()

            last = ci == NFULL

            def grp(g, carry):
                rv = rows_v[p, g, :]
                ebg = plsc.load_gather(btbl, [rv])
                if last:
                    valid = (g * L + iota16) < sz
                    eb = jnp.where(valid, ebg,
                                   jnp.full((L,), B, jnp.int32))
                else:
                    eb = ebg
                rc, lastm = plsc.scan_count(eb)
                plsc.addupdate_scatter(
                    cntl,
                    [lax.shift_right_logical(eb, 4), eb & 15],
                    rc.astype(jnp.float32),
                    mask=lastm,
                )
                for f in range(EOUT):
                    vals = attr_v[p, f, pl.ds(g * L, L)]
                    plsc.addupdate_scatter(
                        acc, [jnp.full((L,), f, jnp.int32), eb], vals)
                return carry
            lax.fori_loop(0, GRPS, grp, 0, unroll=4)

        def trow(r, carry):
            accm[r, :] = plsc.load_gather(
                acc, [iota16, jnp.zeros((L,), jnp.int32) + r])
            return carry
        lax.fori_loop(0, B, trow, 0, unroll=4)

        pltpu.sync_copy(accm.at[pl.ds(0, 128), :], acc_sh.at[i128v.at[0]],
                        add=True)
        pltpu.sync_copy(accm.at[pl.ds(128, 128), :], acc_sh.at[i128v.at[1]],
                        add=True)
        pltpu.sync_copy(cntl, cnt_sh.at[i32v], add=True)
        plsc.subcore_barrier()

        @pl.when(s == 0)
        def _():
            pltpu.sync_copy(acc_sh.at[pl.ds(0, B), :], esum_hbm.at[c])
            pltpu.sync_copy(cnt_sh, ecnt_hbm.at[c])

    return k(row16, attr_t, batch_i)


def _node_agg_tc(x, batch_r):
    nblk = 10
    blk = N // nblk

    def body(batch_ref, x_ref, nsum_ref, ncnt_ref):
        i = pl.program_id(0)
        b = batch_ref[0]
        ohT = (lax.broadcasted_iota(jnp.int32, (B, blk), 0) == b
               ).astype(jnp.float32)
        ns = jnp.dot(ohT, x_ref[...], preferred_element_type=jnp.float32)
        nc = jnp.sum(ohT, axis=1, keepdims=True)

        @pl.when(i == 0)
        def _():
            nsum_ref[...] = jnp.zeros_like(nsum_ref)
            ncnt_ref[...] = jnp.zeros_like(ncnt_ref)

        nsum_ref[...] += ns
        ncnt_ref[...] += nc

    return pl.pallas_call(
        body,
        grid=(nblk,),
        in_specs=[
            pl.BlockSpec((1, 1, blk), lambda i: (i, 0, 0)),
            pl.BlockSpec((blk, NOUT), lambda i: (i, 0)),
        ],
        out_specs=[
            pl.BlockSpec((B, NOUT), lambda i: (0, 0)),
            pl.BlockSpec((B, 1), lambda i: (0, 0)),
        ],
        out_shape=[
            jax.ShapeDtypeStruct((B, NOUT), jnp.float32),
            jax.ShapeDtypeStruct((B, 1), jnp.float32),
        ],
    )(batch_r, x)


def _mlp_tc(u, nsum, ncnt, esum2, ecnt2, W1, b1, gamma, beta, w2, b2):
    uin = u.shape[1]

    def body(u_ref, ns_ref, nc_ref, es_ref, ec_ref, w1_ref, b1_ref,
             g_ref, be_ref, w2_ref, b2_ref, out_ref):
        nmean = ns_ref[...] / jnp.maximum(nc_ref[...], 1.0)
        es = es_ref[0] + es_ref[1]
        ec = ec_ref[0] + ec_ref[1]
        rowsel = (lax.shift_right_logical(
            lax.broadcasted_iota(jnp.int32, (B, 32), 0), 4)
            == lax.broadcasted_iota(jnp.int32, (B, 32), 1)
        ).astype(jnp.float32)
        spread = jnp.dot(rowsel, ec, preferred_element_type=jnp.float32)
        colmask = ((lax.broadcasted_iota(jnp.int32, (B, 16), 0) & 15)
                   == lax.broadcasted_iota(jnp.int32, (B, 16), 1)
                   ).astype(jnp.float32)
        ecnt = jnp.sum(spread * colmask, axis=1, keepdims=True)
        emean = es / jnp.maximum(ecnt, 1.0)
        h = (jnp.dot(u_ref[...], w1_ref[pl.ds(0, uin), :],
                     preferred_element_type=jnp.float32)
             + jnp.dot(nmean, w1_ref[pl.ds(uin, NOUT), :],
                       preferred_element_type=jnp.float32)
             + jnp.dot(emean, w1_ref[pl.ds(uin + NOUT, EOUT), :],
                       preferred_element_type=jnp.float32)
             + b1_ref[...])
        mu = jnp.mean(h, axis=-1, keepdims=True)
        d = h - mu
        var = jnp.mean(d * d, axis=-1, keepdims=True)
        hn = d / jnp.sqrt(var + 1e-5) * g_ref[...] + be_ref[...]
        hr = jnp.maximum(hn, 0.0)
        out_ref[...] = jnp.dot(hr, w2_ref[...],
                               preferred_element_type=jnp.float32) + b2_ref[...]

    return pl.pallas_call(
        body,
        out_shape=jax.ShapeDtypeStruct((B, UOUT), jnp.float32),
    )(u, nsum, ncnt, esum2, ecnt2, W1, b1, gamma, beta, w2, b2)


def kernel(x, edge_index, edge_attr, u, batch, W1, b1, gamma, beta, W2, b2):
    row16 = jnp.maximum(edge_index[0].astype(jnp.int32).reshape(E // L, L),
                        jnp.int32(0))
    batch_i = batch.astype(jnp.int32)
    batch_r = batch_i.reshape(10, 1, N // 10)

    attr_t = edge_attr.T

    nsum, ncnt = _node_agg_tc(x, batch_r)
    esum_p, ecnt_p = _edge_agg_sc(row16, attr_t, batch_i)

    return _mlp_tc(u, nsum, ncnt, esum_p, ecnt_p, W1,
                   b1.reshape(1, HS), gamma.reshape(1, HS),
                   beta.reshape(1, HS), W2, b2.reshape(1, UOUT))


# speedup vs baseline: 1.0085x; 1.0085x over previous
import functools

import jax
import jax.numpy as jnp
from jax import lax
from jax.experimental import pallas as pl
from jax.experimental.pallas import tpu as pltpu
from jax.experimental.pallas import tpu_sc as plsc

N = 10000
E = 320000
B = 256
NOUT = 128
EOUT = 16
HS = 256
UOUT = 128

NC = 2
NS = 16
NW = NC * NS
L = 16

EPT = E // NW
CH = 2048
NFULL = EPT // CH
TAIL = EPT - NFULL * CH
GRPS = CH // L
NBUF = 3


def _edge_agg_sc(row16, attr_t, batch_i):
    mesh = plsc.VectorSubcoreMesh(core_axis_name="c", subcore_axis_name="s")

    @functools.partial(
        pl.kernel,
        out_type=[
            jax.ShapeDtypeStruct((NC, B, EOUT), jnp.float32),
            jax.ShapeDtypeStruct((NC, 32, 16), jnp.float32),
        ],
        mesh=mesh,
        compiler_params=pltpu.CompilerParams(needs_layout_passes=False,
                                             use_tc_tiling_on_sc=False),
        scratch_types=[
            pltpu.VMEM((N,), jnp.int32),
            pltpu.VMEM((NBUF, GRPS, L), jnp.int32),
            pltpu.VMEM((NBUF, EOUT, CH), jnp.float32),
            pltpu.VMEM((EOUT, B + 16), jnp.float32),
            pltpu.VMEM((B + 16, EOUT), jnp.float32),
            pltpu.VMEM((32, 16), jnp.float32),
            pltpu.VMEM((B + 16, EOUT), jnp.float32),
            pltpu.VMEM((32,), jnp.int32),
            pltpu.VMEM((2, 128), jnp.int32),
            [pltpu.SemaphoreType.DMA] * NBUF,
            pltpu.VMEM_SHARED((B + 16, EOUT), jnp.float32),
            pltpu.VMEM_SHARED((32, 16), jnp.float32),
        ],
    )
    def k(row_hbm, attr_hbm, batch_hbm, esum_hbm, ecnt_hbm,
          btbl, rows_v, attr_v, acc, accm, cntl, zbuf, i32v, i128v,
          sl, acc_sh, cnt_sh):
        c = lax.axis_index("c")
        s = lax.axis_index("s")
        wid = s * NC + c
        iota16 = lax.iota(jnp.int32, L)

        def zrow(i, carry):
            zbuf[i, :] = jnp.zeros((L,), jnp.float32)
            return carry
        lax.fori_loop(0, B + 16, zrow, 0)

        def arow(i, carry):
            for f in range(EOUT):
                acc[f, pl.ds(i * L, L)] = jnp.zeros((L,), jnp.float32)
            return carry
        lax.fori_loop(0, (B + 16) // L, arow, 0)

        def crow(i, carry):
            cntl[i, :] = jnp.zeros((L,), jnp.float32)
            return carry
        lax.fori_loop(0, 32, crow, 0)

        i32v[pl.ds(0, 16)] = iota16
        i32v[pl.ds(16, 16)] = iota16 + 16

        def irow(i, carry):
            j = lax.shift_right_logical(i, 3)
            o = (i & 7) * L
            i128v[j, pl.ds(o, L)] = iota16 + i * L
            return carry
        lax.fori_loop(0, 16, irow, 0)

        @pl.when(s == 0)
        def _():
            pltpu.sync_copy(zbuf, acc_sh)
            pltpu.sync_copy(zbuf.at[pl.ds(0, 32), :], cnt_sh)

        pltpu.sync_copy(batch_hbm, btbl)
        plsc.subcore_barrier()

        nchunk = NFULL + 1

        def start_loads(ci):
            p = ci % NBUF
            sz = CH if ci < NFULL else TAIL
            off = wid * EPT + ci * CH
            return [
                pltpu.async_copy(row_hbm.at[pl.ds(off // L, sz // L), :],
                                 rows_v.at[p, pl.ds(0, sz // L), :], sl[p]),
                pltpu.async_copy(attr_hbm.at[:, pl.ds(off, sz)],
                                 attr_v.at[p, :, pl.ds(0, sz)], sl[p]),
            ]

        loads = {0: start_loads(0)}
        for ci in range(nchunk):
            p = ci % NBUF
            sz = CH if ci < NFULL else TAIL
            if ci + 1 < nchunk:
                loads[ci + 1] = start_loads(ci + 1)
            for d in loads.pop(ci):
                d.wait()

            last = ci == NFULL

            def grp(g, carry):
                rv = rows_v[p, g, :]
                ebg = plsc.load_gather(btbl, [rv])
                if last:
                    valid = (g * L + iota16) < sz
                    eb = jnp.where(valid, ebg,
                                   jnp.full((L,), B, jnp.int32))
                else:
                    eb = ebg
                rc, lastm = plsc.scan_count(eb)
                plsc.addupdate_scatter(
                    cntl,
                    [lax.shift_right_logical(eb, 4), eb & 15],
                    rc.astype(jnp.float32),
                    mask=lastm,
                )
                for f in range(EOUT):
                    vals = attr_v[p, f, pl.ds(g * L, L)]
                    plsc.addupdate_scatter(
                        acc, [jnp.full((L,), f, jnp.int32), eb], vals)
                return carry
            lax.fori_loop(0, GRPS, grp, 0, unroll=2)

        def trow(r, carry):
            accm[r, :] = plsc.load_gather(
                acc, [iota16, jnp.zeros((L,), jnp.int32) + r])
            return carry
        lax.fori_loop(0, B, trow, 0, unroll=4)

        pltpu.sync_copy(accm.at[pl.ds(0, 128), :], acc_sh.at[i128v.at[0]],
                        add=True)
        pltpu.sync_copy(accm.at[pl.ds(128, 128), :], acc_sh.at[i128v.at[1]],
                        add=True)
        pltpu.sync_copy(cntl, cnt_sh.at[i32v], add=True)
        plsc.subcore_barrier()

        @pl.when(s == 0)
        def _():
            pltpu.sync_copy(acc_sh.at[pl.ds(0, B), :], esum_hbm.at[c])
            pltpu.sync_copy(cnt_sh, ecnt_hbm.at[c])

    return k(row16, attr_t, batch_i)


def _node_agg_tc(x, batch_r):
    nblk = 10
    blk = N // nblk

    def body(batch_ref, x_ref, nsum_ref, ncnt_ref):
        i = pl.program_id(0)
        b = batch_ref[0]
        ohT = (lax.broadcasted_iota(jnp.int32, (B, blk), 0) == b
               ).astype(jnp.float32)
        ns = jnp.dot(ohT, x_ref[...], preferred_element_type=jnp.float32)
        nc = jnp.sum(ohT, axis=1, keepdims=True)

        @pl.when(i == 0)
        def _():
            nsum_ref[...] = jnp.zeros_like(nsum_ref)
            ncnt_ref[...] = jnp.zeros_like(ncnt_ref)

        nsum_ref[...] += ns
        ncnt_ref[...] += nc

    return pl.pallas_call(
        body,
        grid=(nblk,),
        in_specs=[
            pl.BlockSpec((1, 1, blk), lambda i: (i, 0, 0)),
            pl.BlockSpec((blk, NOUT), lambda i: (i, 0)),
        ],
        out_specs=[
            pl.BlockSpec((B, NOUT), lambda i: (0, 0)),
            pl.BlockSpec((B, 1), lambda i: (0, 0)),
        ],
        out_shape=[
            jax.ShapeDtypeStruct((B, NOUT), jnp.float32),
            jax.ShapeDtypeStruct((B, 1), jnp.float32),
        ],
    )(batch_r, x)


def _mlp_tc(u, nsum, ncnt, esum2, ecnt2, W1, b1, gamma, beta, w2, b2):
    uin = u.shape[1]

    def body(u_ref, ns_ref, nc_ref, es_ref, ec_ref, w1_ref, b1_ref,
             g_ref, be_ref, w2_ref, b2_ref, out_ref):
        nmean = ns_ref[...] / jnp.maximum(nc_ref[...], 1.0)
        es = es_ref[0] + es_ref[1]
        ec = ec_ref[0] + ec_ref[1]
        rowsel = (lax.shift_right_logical(
            lax.broadcasted_iota(jnp.int32, (B, 32), 0), 4)
            == lax.broadcasted_iota(jnp.int32, (B, 32), 1)
        ).astype(jnp.float32)
        spread = jnp.dot(rowsel, ec, preferred_element_type=jnp.float32)
        colmask = ((lax.broadcasted_iota(jnp.int32, (B, 16), 0) & 15)
                   == lax.broadcasted_iota(jnp.int32, (B, 16), 1)
                   ).astype(jnp.float32)
        ecnt = jnp.sum(spread * colmask, axis=1, keepdims=True)
        emean = es / jnp.maximum(ecnt, 1.0)
        h = (jnp.dot(u_ref[...], w1_ref[pl.ds(0, uin), :],
                     preferred_element_type=jnp.float32)
             + jnp.dot(nmean, w1_ref[pl.ds(uin, NOUT), :],
                       preferred_element_type=jnp.float32)
             + jnp.dot(emean, w1_ref[pl.ds(uin + NOUT, EOUT), :],
                       preferred_element_type=jnp.float32)
             + b1_ref[...])
        mu = jnp.mean(h, axis=-1, keepdims=True)
        d = h - mu
        var = jnp.mean(d * d, axis=-1, keepdims=True)
        hn = d / jnp.sqrt(var + 1e-5) * g_ref[...] + be_ref[...]
        hr = jnp.maximum(hn, 0.0)
        out_ref[...] = jnp.dot(hr, w2_ref[...],
                               preferred_element_type=jnp.float32) + b2_ref[...]

    return pl.pallas_call(
        body,
        out_shape=jax.ShapeDtypeStruct((B, UOUT), jnp.float32),
    )(u, nsum, ncnt, esum2, ecnt2, W1, b1, gamma, beta, w2, b2)


def kernel(x, edge_index, edge_attr, u, batch, W1, b1, gamma, beta, W2, b2):
    row16 = jnp.maximum(edge_index[0].astype(jnp.int32).reshape(E // L, L),
                        jnp.int32(0))
    batch_i = batch.astype(jnp.int32)
    batch_r = batch_i.reshape(10, 1, N // 10)

    attr_t = edge_attr.T

    nsum, ncnt = _node_agg_tc(x, batch_r)
    esum_p, ecnt_p = _edge_agg_sc(row16, attr_t, batch_i)

    return _mlp_tc(u, nsum, ncnt, esum_p, ecnt_p, W1,
                   b1.reshape(1, HS), gamma.reshape(1, HS),
                   beta.reshape(1, HS), W2, b2.reshape(1, UOUT))
